# baseline (speedup 1.0000x reference)
import functools

import jax
import jax.numpy as jnp
from jax import lax
from jax.experimental import pallas as pl
from jax.experimental.pallas import tpu as pltpu
from jax.experimental.pallas import tpu_sc as plsc

N = 200000
B = 32
D = 256

NC = 2
NS = 16
NW = NC * NS

CH = 128
NCH = 49
RPW = CH * NCH
NPAD = NW * RPW

_mesh = plsc.VectorSubcoreMesh(core_axis_name="c", subcore_axis_name="s")


@functools.partial(
    pl.kernel,
    out_type=jax.ShapeDtypeStruct((NPAD, D), jnp.float32),
    mesh=_mesh,
    scratch_types=[
        pltpu.VMEM((NCH, CH), jnp.int32),
        pltpu.VMEM((CH, D), jnp.float32),
        pltpu.SemaphoreType.DMA,
    ],
)
def _broadcast_sc(ids_hbm, glob_hbm, out_hbm, idx_v, rows_v, sem):
    wid = lax.axis_index("s") * NC + lax.axis_index("c")
    base_chunk = wid * NCH

    pltpu.sync_copy(ids_hbm.at[wid], idx_v)

    def chunk_body(j, carry):
        pltpu.async_copy(glob_hbm.at[idx_v.at[j]], rows_v, sem).wait()
        pltpu.sync_copy(rows_v, out_hbm.at[pl.ds((base_chunk + j) * CH, CH)])
        return carry

    lax.fori_loop(0, NCH, chunk_body, 0)


def kernel(x, glob, batch_ids):
    ids = batch_ids.astype(jnp.int32)
    ids = jnp.concatenate([ids, jnp.zeros((NPAD - N,), jnp.int32)])
    ids3d = ids.reshape(NW, NCH, CH)
    out = _broadcast_sc(ids3d, glob)
    return out[:N]


# speedup vs baseline: 2.5189x; 2.5189x over previous
import functools

import jax
import jax.numpy as jnp
from jax import lax
from jax.experimental import pallas as pl
from jax.experimental.pallas import tpu as pltpu
from jax.experimental.pallas import tpu_sc as plsc

N = 200000
B = 32
D = 256

NC = 2
NS = 16
NW = NC * NS

RPW = N // NW
CH = 128
NCH = 49
NFULL = RPW // CH
REM = RPW - NFULL * CH
RB = 128

_mesh = plsc.VectorSubcoreMesh(core_axis_name="c", subcore_axis_name="s")


@functools.partial(
    pl.kernel,
    out_type=jax.ShapeDtypeStruct((N * D,), jnp.float32),
    mesh=_mesh,
    scratch_types=[
        pltpu.VMEM((NCH, CH), jnp.int32),
        pltpu.VMEM((B, D), jnp.float32),
        pltpu.VMEM((B, 16), jnp.int32),
        pltpu.VMEM((RB * D,), jnp.float32),
    ],
)
def _broadcast_sc(ids_hbm, glob_hbm, out_hbm, idx_v, glob_v, acc_v, buf):
    wid = lax.axis_index("s") * NC + lax.axis_index("c")

    pltpu.sync_copy(ids_hbm.at[wid], idx_v)
    pltpu.sync_copy(glob_hbm, glob_v)

    zeros = jnp.zeros((16,), jnp.int32)
    ones = jnp.ones((16,), jnp.int32)
    lane = lax.iota(jnp.int32, 16)
    tail_mask = lane < (REM - 96)

    for t in range(B):
        acc_v[t, :] = zeros

    def hist_row(j, carry):
        vs = [idx_v[j, pl.ds(k * 16, 16)] for k in range(8)]
        for t in range(B):
            a = acc_v[t, :]
            for k in range(8):
                a = a + jnp.where(vs[k] == t, ones, zeros)
            acc_v[t, :] = a
        return carry

    lax.fori_loop(0, NFULL, hist_row, 0)
    vs = [idx_v[NFULL, pl.ds(k * 16, 16)] for k in range(7)]
    for t in range(B):
        a = acc_v[t, :]
        for k in range(6):
            a = a + jnp.where(vs[k] == t, ones, zeros)
        a = a + jnp.where(tail_mask & (vs[6] == t), ones, zeros)
        acc_v[t, :] = a

    out_base = wid * (RPW * D)

    def emit(b, off):
        a = acc_v[b, :]
        cnt = a[0]
        for l in range(1, 16):
            cnt = cnt + a[l]

        @pl.when(cnt > 0)
        def _():
            def fill_row(r, c2):
                for c in range(D // 16):
                    buf[pl.ds(r * D + c * 16, 16)] = glob_v[b, pl.ds(c * 16, 16)]
                return c2

            lax.fori_loop(0, RB, fill_row, 0)

            def dma_full(i, o):
                pltpu.sync_copy(buf.at[pl.ds(0, RB * D)],
                                out_hbm.at[pl.ds(pl.multiple_of(o, D), RB * D)])
                return o + RB * D

            o = lax.fori_loop(0, cnt // RB, dma_full, off)

            for sz in (64, 32, 16, 8, 4, 2, 1):
                @pl.when((cnt & sz) != 0)
                def _(sz=sz, o=o):
                    pltpu.sync_copy(buf.at[pl.ds(0, sz * D)],
                                    out_hbm.at[pl.ds(pl.multiple_of(o, D), sz * D)])
                o = o + (cnt & sz) * D

        return off + cnt * D

    lax.fori_loop(0, B, emit, out_base)


def kernel(x, glob, batch_ids):
    ids = batch_ids.astype(jnp.int32)
    pos = (jnp.arange(NW, dtype=jnp.int32) * RPW)[:, None] + jnp.arange(
        NCH * CH, dtype=jnp.int32)[None, :]
    ids_r = jnp.take(ids, jnp.minimum(pos, N - 1)).reshape(NW, NCH, CH)
    out = _broadcast_sc(ids_r, glob)
    return out.reshape(N, D)
